# baseline (speedup 1.0000x reference)
import functools

import jax
import jax.numpy as jnp
from jax import lax
from jax.experimental import pallas as pl
from jax.experimental.pallas import tpu as pltpu
from jax.experimental.pallas import tpu_sc as plsc

_D = 64
_W = 2 * _D
_NC = 2
_NS = 16
_NW = _NC * _NS
_L = 16
_BB = 2048


def _sc_gather_body(id_hbm, t_hbm, out_hbm, idx_v, rows_v, sem, *, bpw, rpp):
    wid = lax.axis_index("s") * _NC + lax.axis_index("c")
    base = wid * bpw
    pltpu.sync_copy(id_hbm.at[pl.ds(base, bpw)], idx_v)

    for p in range(bpw // rpp):
        def group(g, _):
            v = idx_v[pl.ds(p * rpp + g * _L, _L)]
            for j in range(_L):
                pltpu.async_copy(t_hbm.at[pl.ds(v[j], 1)],
                                 rows_v.at[pl.ds(g * _L + j, 1)], sem)
            return ()

        lax.fori_loop(0, rpp // _L, group, (), unroll=False)
        pltpu.make_async_copy(t_hbm.at[pl.ds(0, rpp)], rows_v, sem).wait()
        pltpu.sync_copy(rows_v, out_hbm.at[pl.ds(base + p * rpp, rpp)])


def _sc_gather(ids, pairs):
    batch = ids.shape[0]
    bpw = batch // _NW
    rpp = min(bpw, 256)
    row_t = jax.ShapeDtypeStruct((batch, _W), jnp.float32)
    k = pl.kernel(
        functools.partial(_sc_gather_body, bpw=bpw, rpp=rpp),
        mesh=plsc.VectorSubcoreMesh(core_axis_name="c", subcore_axis_name="s"),
        compiler_params=pltpu.CompilerParams(use_tc_tiling_on_sc=True),
        out_type=[row_t],
        scratch_types=[
            pltpu.VMEM((bpw,), jnp.int32),
            pltpu.VMEM((rpp, _W), jnp.float32),
            pltpu.SemaphoreType.DMA,
        ],
    )
    return k(ids, pairs)[0]


def _tp_body(a_ref, b_ref, o_ref):
    o_ref[...] = jnp.concatenate([a_ref[...].T, b_ref[...].T], axis=1)


_TBLK = 8192


def _transpose_pack(t):
    n = t.shape[1]
    grid = (n + 2 * _TBLK - 1) // (2 * _TBLK)
    out_t = jax.ShapeDtypeStruct((grid * _TBLK, _W), jnp.float32)
    last = (n + _TBLK - 1) // _TBLK - 1
    even = lambda j: (0, jnp.minimum(2 * j, last))
    odd = lambda j: (0, jnp.minimum(2 * j + 1, last))
    return pl.pallas_call(
        _tp_body,
        grid=(grid,),
        in_specs=[
            pl.BlockSpec((_D, _TBLK), even),
            pl.BlockSpec((_D, _TBLK), odd),
        ],
        out_specs=pl.BlockSpec((_TBLK, _W), lambda j: (j, 0)),
        out_shape=out_t,
        compiler_params=pltpu.CompilerParams(
            dimension_semantics=("arbitrary",)),
    )(t, t)


def _mlp_body(uw_ref, iw_ref, uo_ref, io_ref, w1u_ref, w1i_ref, b1_ref,
              w2_ref, b2_ref, w3_ref, b3_ref, w4t_ref, b4_ref, o_ref):
    uw = uw_ref[...]
    iw = iw_ref[...]
    u = jnp.where(uo_ref[...] == 1, uw[:, _D:], uw[:, :_D]).astype(jnp.float32)
    i = jnp.where(io_ref[...] == 1, iw[:, _D:], iw[:, :_D]).astype(jnp.float32)
    h = jnp.dot(u, w1u_ref[...], preferred_element_type=jnp.float32)
    h = h + jnp.dot(i, w1i_ref[...], preferred_element_type=jnp.float32)
    h = jnp.maximum(h + b1_ref[...], 0.0)
    h = jnp.maximum(
        jnp.dot(h, w2_ref[...], preferred_element_type=jnp.float32) + b2_ref[...], 0.0)
    h = jnp.maximum(
        jnp.dot(h, w3_ref[...], preferred_element_type=jnp.float32) + b3_ref[...], 0.0)
    o_ref[...] = jnp.sum(h * w4t_ref[...], axis=1, keepdims=True) + b4_ref[...]


def kernel(user_ids, item_ids, user_table, item_table,
           W1, b1, W2, b2, W3, b3, W4, b4):
    batch = user_ids.shape[0]
    n_rows = user_table.shape[0]
    uids = user_ids.astype(jnp.int32)
    iids = item_ids.astype(jnp.int32)

    del n_rows
    upos = ((uids >> 14) << 13) | (uids & (_TBLK - 1))
    ipos = ((iids >> 14) << 13) | (iids & (_TBLK - 1))
    u2 = _transpose_pack(user_table.T)
    uw = _sc_gather(upos, u2)
    i2 = _transpose_pack(item_table.T)
    iw = _sc_gather(ipos, i2)
    uo = ((uids >> 13) & 1).reshape(batch, 1)
    io = ((iids >> 13) & 1).reshape(batch, 1)

    w1u = W1[:_D]
    w1i = W1[_D:]
    b1r = b1.reshape(1, -1)
    b2r = b2.reshape(1, -1)
    b3r = b3.reshape(1, -1)
    w4t = W4.reshape(1, -1)
    b4r = b4.reshape(1, 1)

    out = pl.pallas_call(
        _mlp_body,
        grid=(batch // _BB,),
        in_specs=[
            pl.BlockSpec((_BB, _W), lambda b: (b, 0)),
            pl.BlockSpec((_BB, _W), lambda b: (b, 0)),
            pl.BlockSpec((_BB, 1), lambda b: (b, 0)),
            pl.BlockSpec((_BB, 1), lambda b: (b, 0)),
            pl.BlockSpec(w1u.shape, lambda b: (0, 0)),
            pl.BlockSpec(w1i.shape, lambda b: (0, 0)),
            pl.BlockSpec(b1r.shape, lambda b: (0, 0)),
            pl.BlockSpec(W2.shape, lambda b: (0, 0)),
            pl.BlockSpec(b2r.shape, lambda b: (0, 0)),
            pl.BlockSpec(W3.shape, lambda b: (0, 0)),
            pl.BlockSpec(b3r.shape, lambda b: (0, 0)),
            pl.BlockSpec(w4t.shape, lambda b: (0, 0)),
            pl.BlockSpec(b4r.shape, lambda b: (0, 0)),
        ],
        out_specs=pl.BlockSpec((_BB, 1), lambda b: (b, 0)),
        out_shape=jax.ShapeDtypeStruct((batch, 1), jnp.float32),
        compiler_params=pltpu.CompilerParams(
            dimension_semantics=("arbitrary",)),
    )(uw, iw, uo, io, w1u, w1i, b1r, W2, b2r, W3, b3r, w4t, b4r)
    return out[:, 0]


# speedup vs baseline: 1.0681x; 1.0681x over previous
import functools

import jax
import jax.numpy as jnp
from jax import lax
from jax.experimental import pallas as pl
from jax.experimental.pallas import tpu as pltpu
from jax.experimental.pallas import tpu_sc as plsc

_D = 64
_W = 2 * _D
_NC = 2
_NS = 16
_NW = _NC * _NS
_L = 16
_BB = 2048


def _sc_gather_body(uid_hbm, iid_hbm, ut_hbm, it_hbm, u_out, i_out,
                    uidx_v, iidx_v, urows_v, irows_v, sem, *, bpw, rpp):
    wid = lax.axis_index("s") * _NC + lax.axis_index("c")
    base = wid * bpw
    pltpu.sync_copy(uid_hbm.at[pl.ds(base, bpw)], uidx_v)
    pltpu.sync_copy(iid_hbm.at[pl.ds(base, bpw)], iidx_v)

    for p in range(bpw // rpp):
        def group(g, _):
            uv = uidx_v[pl.ds(p * rpp + g * _L, _L)]
            iv = iidx_v[pl.ds(p * rpp + g * _L, _L)]
            for j in range(_L):
                pltpu.async_copy(ut_hbm.at[pl.ds(uv[j], 1)],
                                 urows_v.at[pl.ds(g * _L + j, 1)], sem)
                pltpu.async_copy(it_hbm.at[pl.ds(iv[j], 1)],
                                 irows_v.at[pl.ds(g * _L + j, 1)], sem)
            return ()

        lax.fori_loop(0, rpp // _L, group, (), unroll=False)
        pltpu.make_async_copy(ut_hbm.at[pl.ds(0, rpp)], urows_v, sem).wait()
        pltpu.make_async_copy(it_hbm.at[pl.ds(0, rpp)], irows_v, sem).wait()
        pltpu.sync_copy(urows_v, u_out.at[pl.ds(base + p * rpp, rpp)])
        pltpu.sync_copy(irows_v, i_out.at[pl.ds(base + p * rpp, rpp)])


def _sc_gather(uids, iids, user_pairs, item_pairs):
    batch = uids.shape[0]
    bpw = batch // _NW
    rpp = min(bpw, 128)
    row_t = jax.ShapeDtypeStruct((batch, _W), jnp.float32)
    k = pl.kernel(
        functools.partial(_sc_gather_body, bpw=bpw, rpp=rpp),
        mesh=plsc.VectorSubcoreMesh(core_axis_name="c", subcore_axis_name="s"),
        compiler_params=pltpu.CompilerParams(use_tc_tiling_on_sc=True),
        out_type=[row_t, row_t],
        scratch_types=[
            pltpu.VMEM((bpw,), jnp.int32),
            pltpu.VMEM((bpw,), jnp.int32),
            pltpu.VMEM((rpp, _W), jnp.float32),
            pltpu.VMEM((rpp, _W), jnp.float32),
            pltpu.SemaphoreType.DMA,
        ],
    )
    return k(uids, iids, user_pairs, item_pairs)


def _tp_body(ua_ref, ub_ref, ia_ref, ib_ref, u2_ref, i2_ref):
    u2_ref[...] = jnp.concatenate(
        [ua_ref[...].T, ub_ref[...].T], axis=1)
    i2_ref[...] = jnp.concatenate(
        [ia_ref[...].T, ib_ref[...].T], axis=1)


_TBLK = 8192


def _transpose_pack(ut, it):
    n = ut.shape[1]
    grid = (n + 2 * _TBLK - 1) // (2 * _TBLK)
    out_t = jax.ShapeDtypeStruct((grid * _TBLK, _W), jnp.float32)
    last = (n + _TBLK - 1) // _TBLK - 1
    even = lambda j: (0, jnp.minimum(2 * j, last))
    odd = lambda j: (0, jnp.minimum(2 * j + 1, last))
    return pl.pallas_call(
        _tp_body,
        grid=(grid,),
        in_specs=[
            pl.BlockSpec((_D, _TBLK), even),
            pl.BlockSpec((_D, _TBLK), odd),
            pl.BlockSpec((_D, _TBLK), even),
            pl.BlockSpec((_D, _TBLK), odd),
        ],
        out_specs=[
            pl.BlockSpec((_TBLK, _W), lambda j: (j, 0)),
            pl.BlockSpec((_TBLK, _W), lambda j: (j, 0)),
        ],
        out_shape=[out_t, out_t],
        compiler_params=pltpu.CompilerParams(
            dimension_semantics=("arbitrary",)),
    )(ut, ut, it, it)


def _mlp_body(uw_ref, iw_ref, uo_ref, io_ref, w1u_ref, w1i_ref, b1_ref,
              w2_ref, b2_ref, w3_ref, b3_ref, w4t_ref, b4_ref, o_ref):
    uw = uw_ref[...]
    iw = iw_ref[...]
    u = jnp.where(uo_ref[...] == 1, uw[:, _D:], uw[:, :_D]).astype(jnp.float32)
    i = jnp.where(io_ref[...] == 1, iw[:, _D:], iw[:, :_D]).astype(jnp.float32)
    h = jnp.dot(u, w1u_ref[...], preferred_element_type=jnp.float32)
    h = h + jnp.dot(i, w1i_ref[...], preferred_element_type=jnp.float32)
    h = jnp.maximum(h + b1_ref[...], 0.0)
    h = jnp.maximum(
        jnp.dot(h, w2_ref[...], preferred_element_type=jnp.float32) + b2_ref[...], 0.0)
    h = jnp.maximum(
        jnp.dot(h, w3_ref[...], preferred_element_type=jnp.float32) + b3_ref[...], 0.0)
    o_ref[...] = jnp.sum(h * w4t_ref[...], axis=1, keepdims=True) + b4_ref[...]


def kernel(user_ids, item_ids, user_table, item_table,
           W1, b1, W2, b2, W3, b3, W4, b4):
    batch = user_ids.shape[0]
    n_rows = user_table.shape[0]
    uids = user_ids.astype(jnp.int32)
    iids = item_ids.astype(jnp.int32)

    del n_rows
    u2, i2 = _transpose_pack(user_table.T, item_table.T)

    upos = ((uids >> 14) << 13) | (uids & (_TBLK - 1))
    ipos = ((iids >> 14) << 13) | (iids & (_TBLK - 1))
    uw, iw = _sc_gather(upos, ipos, u2, i2)
    uo = ((uids >> 13) & 1).reshape(batch, 1)
    io = ((iids >> 13) & 1).reshape(batch, 1)

    w1u = W1[:_D]
    w1i = W1[_D:]
    b1r = b1.reshape(1, -1)
    b2r = b2.reshape(1, -1)
    b3r = b3.reshape(1, -1)
    w4t = W4.reshape(1, -1)
    b4r = b4.reshape(1, 1)

    out = pl.pallas_call(
        _mlp_body,
        grid=(batch // _BB,),
        in_specs=[
            pl.BlockSpec((_BB, _W), lambda b: (b, 0)),
            pl.BlockSpec((_BB, _W), lambda b: (b, 0)),
            pl.BlockSpec((_BB, 1), lambda b: (b, 0)),
            pl.BlockSpec((_BB, 1), lambda b: (b, 0)),
            pl.BlockSpec(w1u.shape, lambda b: (0, 0)),
            pl.BlockSpec(w1i.shape, lambda b: (0, 0)),
            pl.BlockSpec(b1r.shape, lambda b: (0, 0)),
            pl.BlockSpec(W2.shape, lambda b: (0, 0)),
            pl.BlockSpec(b2r.shape, lambda b: (0, 0)),
            pl.BlockSpec(W3.shape, lambda b: (0, 0)),
            pl.BlockSpec(b3r.shape, lambda b: (0, 0)),
            pl.BlockSpec(w4t.shape, lambda b: (0, 0)),
            pl.BlockSpec(b4r.shape, lambda b: (0, 0)),
        ],
        out_specs=pl.BlockSpec((_BB, 1), lambda b: (b, 0)),
        out_shape=jax.ShapeDtypeStruct((batch, 1), jnp.float32),
        compiler_params=pltpu.CompilerParams(
            dimension_semantics=("arbitrary",)),
    )(uw, iw, uo, io, w1u, w1i, b1r, W2, b2r, W3, b3r, w4t, b4r)
    return out[:, 0]
